# baseline (device time: 64021 ns/iter reference)
import jax
import jax.numpy as jnp
from jax import lax
from jax.experimental import pallas as pl
from jax.experimental.pallas import tpu as pltpu

C = 16


def kernel(x):
    m, n = x.shape
    h = m // 2
    r = h // C

    def body(x_hbm, out_hbm, x_my, x_oth, send_y, recv_y, recv_x, out_buf,
             lm_sems, lo_sems, om_sems, oo_sems,
             ys_sems, yr_sems, xs_sems, xr_sems):
        my_x = lax.axis_index("x")
        my_y = lax.axis_index("y")
        my_z = lax.axis_index("z")
        y_nbr = (my_x, 1 - my_y, my_z)
        x_nbr = (1 - my_x, my_y, my_z)

        my_off = my_x * h
        other_off = (1 - my_x) * h

        in_my = []
        for c in range(C):
            cp = pltpu.make_async_copy(
                x_hbm.at[pl.ds(my_off + c * r, r)],
                x_my.at[pl.ds(c * r, r)],
                lm_sems.at[c],
            )
            cp.start()
            in_my.append(cp)
        in_oth = []
        for c in range(C):
            cp = pltpu.make_async_copy(
                x_hbm.at[pl.ds(other_off + c * r, r)],
                x_oth.at[pl.ds(c * r, r)],
                lo_sems.at[c],
            )
            cp.start()
            in_oth.append(cp)

        barrier_sem = pltpu.get_barrier_semaphore()
        for nbr in (y_nbr, x_nbr):
            pl.semaphore_signal(
                barrier_sem, inc=1,
                device_id=nbr, device_id_type=pl.DeviceIdType.MESH,
            )
        pl.semaphore_wait(barrier_sem, 2)

        y_rdmas = []
        for c in range(C):
            in_my[c].wait()
            send_y[pl.ds(c * r, r), :] = (
                x_my[pl.ds(c * r, r), :].astype(jnp.bfloat16)
            )
            rd = pltpu.make_async_remote_copy(
                src_ref=send_y.at[pl.ds(c * r, r)],
                dst_ref=recv_y.at[pl.ds(c * r, r)],
                send_sem=ys_sems.at[c],
                recv_sem=yr_sems.at[c],
                device_id=y_nbr,
                device_id_type=pl.DeviceIdType.MESH,
            )
            rd.start()
            y_rdmas.append(rd)

        x_rdmas = []
        out_cps = []
        for c in range(C):
            y_rdmas[c].wait_recv()
            rd = pltpu.make_async_remote_copy(
                src_ref=recv_y.at[pl.ds(c * r, r)],
                dst_ref=recv_x.at[pl.ds(c * r, r)],
                send_sem=xs_sems.at[c],
                recv_sem=xr_sems.at[c],
                device_id=x_nbr,
                device_id_type=pl.DeviceIdType.MESH,
            )
            rd.start()
            x_rdmas.append(rd)
            out_buf[pl.ds(my_off + c * r, r), :] = (
                x_my[pl.ds(c * r, r), :].astype(jnp.bfloat16)
                + recv_y[pl.ds(c * r, r), :]
            )
            cp = pltpu.make_async_copy(
                out_buf.at[pl.ds(my_off + c * r, r)],
                out_hbm.at[pl.ds(my_off + c * r, r)],
                om_sems.at[c],
            )
            cp.start()
            out_cps.append(cp)

        for c in range(C):
            x_rdmas[c].wait_recv()
            in_oth[c].wait()
            out_buf[pl.ds(other_off + c * r, r), :] = (
                x_oth[pl.ds(c * r, r), :].astype(jnp.bfloat16)
                + recv_x[pl.ds(c * r, r), :]
            )
            cp = pltpu.make_async_copy(
                out_buf.at[pl.ds(other_off + c * r, r)],
                out_hbm.at[pl.ds(other_off + c * r, r)],
                oo_sems.at[c],
            )
            cp.start()
            out_cps.append(cp)

        for c in range(C):
            y_rdmas[c].wait_send()
            x_rdmas[c].wait_send()
        for cp in out_cps:
            cp.wait()

    return pl.pallas_call(
        body,
        out_shape=jax.ShapeDtypeStruct((m, n), jnp.bfloat16),
        in_specs=[pl.BlockSpec(memory_space=pl.ANY)],
        out_specs=pl.BlockSpec(memory_space=pl.ANY),
        scratch_shapes=[
            pltpu.VMEM((h, n), jnp.float32),
            pltpu.VMEM((h, n), jnp.float32),
            pltpu.VMEM((h, n), jnp.bfloat16),
            pltpu.VMEM((h, n), jnp.bfloat16),
            pltpu.VMEM((h, n), jnp.bfloat16),
            pltpu.VMEM((m, n), jnp.bfloat16),
            pltpu.SemaphoreType.DMA((C,)),
            pltpu.SemaphoreType.DMA((C,)),
            pltpu.SemaphoreType.DMA((C,)),
            pltpu.SemaphoreType.DMA((C,)),
            pltpu.SemaphoreType.DMA((C,)),
            pltpu.SemaphoreType.DMA((C,)),
            pltpu.SemaphoreType.DMA((C,)),
            pltpu.SemaphoreType.DMA((C,)),
        ],
        compiler_params=pltpu.CompilerParams(
            collective_id=0, vmem_limit_bytes=64 * 1024 * 1024
        ),
    )(x)
